# Initial kernel scaffold; baseline (speedup 1.0000x reference)
import jax
import jax.numpy as jnp
from jax.experimental import pallas as pl


def kernel(query, p3, p4, p5, Wref, bref, Woff, boff, Wattn, battn, Wv, bv, Wout, bout, W1, b1, W2, b2, g1, be1, g2, be2):
    raise NotImplementedError("write your pallas kernel here")


# speedup vs baseline: 49.1179x; 49.1179x over previous
import functools

import numpy as np
import jax
import jax.numpy as jnp
from jax import lax
from jax.experimental import pallas as pl
from jax.experimental.pallas import tpu as pltpu
from jax.experimental.pallas import tpu_sc as plsc

B_ = 2
D_ = 256
H_ = 8
HD_ = 32
L_ = 3
P_ = 4
S_ = L_ * P_
K_ = S_ * 4
DFF_ = 1024
SHAPES_ = ((64, 64), (32, 32), (16, 16))
NQ_ = sum(h * w for h, w in SHAPES_)
NV_ = NQ_
ROWS_ = B_ * NQ_

RB_ = 448
QC_ = 256

_f32 = jnp.float32
_i32 = jnp.int32


def _lane_consts():
    l_of = np.tile(np.repeat(np.arange(L_), P_), H_)
    wl = np.array([SHAPES_[l][1] for l in l_of])
    hl = np.array([SHAPES_[l][0] for l in l_of])
    bases = np.concatenate([[0], np.cumsum([h * w for h, w in SHAPES_])[:-1]])
    base = np.array([bases[l] for l in l_of])
    return (
        jnp.asarray(wl[None, :], _f32),
        jnp.asarray(hl[None, :], _f32),
        jnp.asarray(1.0 / wl[None, :], _f32),
        jnp.asarray(1.0 / hl[None, :], _f32),
        jnp.asarray((wl - 1)[None, :], _i32),
        jnp.asarray((hl - 1)[None, :], _i32),
        jnp.asarray(base[None, :], _i32),
    )


def _softmax_mat():
    g = np.arange(96) // S_
    return jnp.asarray((g[:, None] == g[None, :]).astype(np.float32))


def _plan_body(q_ref, w_ref, b_ref, m_ref, cw_ref, ch_ref, ciw_ref, cih_ref,
               wmax_ref, hmax_ref, base_ref,
               i00, i01, i10, i11, w00, w01, w10, w11):
    t = jnp.dot(q_ref[...], w_ref[...], preferred_element_type=_f32) + b_ref[...]
    logits = t[:, 0:96]
    offx = t[:, 128:224]
    offy = t[:, 256:352]
    refx = jax.nn.sigmoid(t[:, 384:385])
    refy = jax.nn.sigmoid(t[:, 385:386])

    e = jnp.exp(logits)
    den = jnp.dot(e, m_ref[...], preferred_element_type=_f32)
    aw = e / den

    x = (refx + offx * ciw_ref[...]) * cw_ref[...] - 0.5
    y = (refy + offy * cih_ref[...]) * ch_ref[...] - 0.5
    x0f = jnp.floor(x)
    y0f = jnp.floor(y)
    wx = x - x0f
    wy = y - y0f
    wmax = wmax_ref[...]
    hmax = hmax_ref[...]
    x0 = jnp.clip(x0f.astype(_i32), 0, wmax)
    x1 = jnp.clip(x0f.astype(_i32) + 1, 0, wmax)
    y0 = jnp.clip(y0f.astype(_i32), 0, hmax)
    y1 = jnp.clip(y0f.astype(_i32) + 1, 0, hmax)
    wl_i = wmax + 1
    base = base_ref[...]
    i00[...] = (y0 * wl_i + x0 + base) * 16
    i01[...] = (y0 * wl_i + x1 + base) * 16
    i10[...] = (y1 * wl_i + x0 + base) * 16
    i11[...] = (y1 * wl_i + x1 + base) * 16
    w00[...] = (1.0 - wy) * (1.0 - wx) * aw
    w01[...] = (1.0 - wy) * wx * aw
    w10[...] = wy * (1.0 - wx) * aw
    w11[...] = wy * wx * aw


def _plan_call(q2d, wcat, bcat):
    nb = ROWS_ // RB_
    consts = _lane_consts()
    m = _softmax_mat()
    full = lambda shp: pl.BlockSpec(shp, lambda i: (0, 0))
    out_sh = ([jax.ShapeDtypeStruct((ROWS_, 96), _i32)] * 4
              + [jax.ShapeDtypeStruct((ROWS_, 96), _f32)] * 4)
    return pl.pallas_call(
        _plan_body,
        grid=(nb,),
        in_specs=[pl.BlockSpec((RB_, D_), lambda i: (i, 0)),
                  full((D_, 512)), full((1, 512)), full((96, 96)),
                  full((1, 96)), full((1, 96)), full((1, 96)), full((1, 96)),
                  full((1, 96)), full((1, 96)), full((1, 96))],
        out_specs=[pl.BlockSpec((RB_, 96), lambda i: (i, 0))] * 8,
        out_shape=out_sh,
    )(q2d, wcat, bcat, m, *consts)


def _vproj_body(f_ref, w_ref, b_ref, o_ref):
    o_ref[...] = (jnp.dot(f_ref[...], w_ref[...], preferred_element_type=_f32)
                  + b_ref[...])


def _vproj_call(feat2d, wv, bv):
    nb = ROWS_ // RB_
    return pl.pallas_call(
        _vproj_body,
        grid=(nb,),
        in_specs=[pl.BlockSpec((RB_, D_), lambda i: (i, 0)),
                  pl.BlockSpec((D_, D_), lambda i: (0, 0)),
                  pl.BlockSpec((1, D_), lambda i: (0, 0))],
        out_specs=pl.BlockSpec((RB_, D_), lambda i: (i, 0)),
        out_shape=jax.ShapeDtypeStruct((ROWS_, D_), _f32),
    )(feat2d, wv, bv)


def _sc_gather(vt, idx_a, wgt_a, interpret=False):
    mesh = plsc.VectorSubcoreMesh(core_axis_name="c", subcore_axis_name="s")
    nch = NQ_ // QC_

    @functools.partial(
        pl.kernel,
        out_type=jax.ShapeDtypeStruct((B_, H_, 2, 16, NQ_), _f32),
        mesh=mesh,
        interpret=interpret,
        compiler_params=pltpu.CompilerParams(needs_layout_passes=False),
        scratch_types=[
            pltpu.VMEM((NV_ * 16,), _f32),
            pltpu.VMEM((K_, QC_), _i32),
            pltpu.VMEM((K_, QC_), _f32),
            pltpu.VMEM((16, QC_), _f32),
        ],
    )
    def sc_fn(vt_h, idx_h, wgt_h, out_h, table_v, idx_v, wgt_v, out_v):
        cid = lax.axis_index("c")
        sid = lax.axis_index("s")
        wid = cid * 16 + sid
        b = wid // 16
        r = wid % 16
        h = r // 2
        half = r % 2
        pltpu.sync_copy(vt_h.at[b, h, half], table_v)

        def chunk_body(ci, carry):
            qo = ci * QC_
            pltpu.sync_copy(idx_h.at[b, h, :, pl.ds(qo, QC_)], idx_v)
            pltpu.sync_copy(wgt_h.at[b, h, :, pl.ds(qo, QC_)], wgt_v)

            def g_body(g, c2):
                go = g * 16
                accs = [jnp.zeros((16,), _f32) for _ in range(16)]
                for k in range(K_):
                    iv = idx_v[k, pl.ds(go, 16)]
                    wv = wgt_v[k, pl.ds(go, 16)]
                    for ch in range(16):
                        val = plsc.load_gather(table_v, [iv + ch])
                        accs[ch] = accs[ch] + wv * val
                for ch in range(16):
                    out_v[ch, pl.ds(go, 16)] = accs[ch]
                return c2

            lax.fori_loop(0, QC_ // 16, g_body, 0)
            pltpu.sync_copy(out_v, out_h.at[b, h, half, :, pl.ds(qo, QC_)])
            return carry

        lax.fori_loop(0, nch, chunk_body, 0)

    return sc_fn(vt, idx_a, wgt_a)


def _tail_body(a_ref, q_ref, wout_ref, bout_ref, g1_ref, be1_ref,
               w1_ref, b1_ref, w2_ref, b2_ref, g2_ref, be2_ref, o_ref):
    a = (jnp.dot(a_ref[...], wout_ref[...], preferred_element_type=_f32)
         + bout_ref[...] + q_ref[...])
    m = jnp.mean(a, axis=-1, keepdims=True)
    v = jnp.mean((a - m) * (a - m), axis=-1, keepdims=True)
    x1 = (a - m) / jnp.sqrt(v + 1e-5) * g1_ref[...] + be1_ref[...]
    hpre = jnp.dot(x1, w1_ref[...], preferred_element_type=_f32) + b1_ref[...]
    hact = 0.5 * hpre * (1.0 + lax.erf(hpre * np.float32(1.0 / np.sqrt(2.0))))
    yv = jnp.dot(hact, w2_ref[...], preferred_element_type=_f32) + b2_ref[...] + x1
    m2 = jnp.mean(yv, axis=-1, keepdims=True)
    v2 = jnp.mean((yv - m2) * (yv - m2), axis=-1, keepdims=True)
    o_ref[...] = (yv - m2) / jnp.sqrt(v2 + 1e-5) * g2_ref[...] + be2_ref[...]


def _tail_call(attn2d, q2d, wout, bout, g1, be1, w1, b1, w2, b2, g2, be2):
    nb = ROWS_ // RB_
    full = lambda shp: pl.BlockSpec(shp, lambda i: (0, 0))
    return pl.pallas_call(
        _tail_body,
        grid=(nb,),
        in_specs=[pl.BlockSpec((RB_, D_), lambda i: (i, 0)),
                  pl.BlockSpec((RB_, D_), lambda i: (i, 0)),
                  full((D_, D_)), full((1, D_)), full((1, D_)), full((1, D_)),
                  full((D_, DFF_)), full((1, DFF_)),
                  full((DFF_, D_)), full((1, D_)), full((1, D_)), full((1, D_))],
        out_specs=pl.BlockSpec((RB_, D_), lambda i: (i, 0)),
        out_shape=jax.ShapeDtypeStruct((ROWS_, D_), _f32),
    )(attn2d, q2d, wout, bout, g1, be1, w1, b1, w2, b2, g2, be2)


def _build_wcat(Wref, bref, Woff, boff, Wattn, battn):
    wcat = jnp.zeros((D_, 512), _f32)
    bcat = jnp.zeros((1, 512), _f32)
    wcat = wcat.at[:, 0:96].set(Wattn)
    bcat = bcat.at[0, 0:96].set(battn)
    woff_r = Woff.reshape(D_, H_, L_, P_, 2)
    boff_r = boff.reshape(H_, L_, P_, 2)
    wcat = wcat.at[:, 128:224].set(woff_r[..., 0].reshape(D_, 96))
    bcat = bcat.at[0, 128:224].set(boff_r[..., 0].reshape(96))
    wcat = wcat.at[:, 256:352].set(woff_r[..., 1].reshape(D_, 96))
    bcat = bcat.at[0, 256:352].set(boff_r[..., 1].reshape(96))
    wcat = wcat.at[:, 384:386].set(Wref)
    bcat = bcat.at[0, 384:386].set(bref)
    return wcat, bcat


def kernel(query, p3, p4, p5, Wref, bref, Woff, boff, Wattn, battn, Wv, bv,
           Wout, bout, W1, b1, W2, b2, g1, be1, g2, be2):
    q2d = query.reshape(ROWS_, D_)
    wcat, bcat = _build_wcat(Wref, bref, Woff, boff, Wattn, battn)

    plan_out = _plan_call(q2d, wcat, bcat)
    i_corners = plan_out[:4]
    w_corners = plan_out[4:]

    def to_sc(arrs):
        st = jnp.stack(arrs, axis=-1)
        st = st.reshape(B_, NQ_, H_, S_, 4)
        st = st.transpose(0, 2, 3, 4, 1)
        return st.reshape(B_, H_, K_, NQ_)

    idx_a = to_sc(i_corners)
    wgt_a = to_sc(w_corners)

    feat2d = jnp.concatenate([p3, p4, p5], axis=1).reshape(ROWS_, D_)
    v2d = _vproj_call(feat2d, Wv, bv.reshape(1, D_))
    vt = (v2d.reshape(B_, NV_, H_, 2, 16).transpose(0, 2, 3, 1, 4)
          .reshape(B_, H_, 2, NV_ * 16))

    attn = _sc_gather(vt, idx_a, wgt_a)
    attn2d = attn.transpose(0, 4, 1, 2, 3).reshape(ROWS_, D_)

    out2d = _tail_call(attn2d, q2d, Wout, bout.reshape(1, D_),
                       g1.reshape(1, D_), be1.reshape(1, D_),
                       W1, b1.reshape(1, DFF_), W2, b2.reshape(1, D_),
                       g2.reshape(1, D_), be2.reshape(1, D_))
    return out2d.reshape(B_, NQ_, D_)


# speedup vs baseline: 69.1350x; 1.4075x over previous
import functools

import numpy as np
import jax
import jax.numpy as jnp
from jax import lax
from jax.experimental import pallas as pl
from jax.experimental.pallas import tpu as pltpu
from jax.experimental.pallas import tpu_sc as plsc

B_ = 2
D_ = 256
H_ = 8
HD_ = 32
L_ = 3
P_ = 4
S_ = L_ * P_
K_ = S_ * 4
DFF_ = 1024
SHAPES_ = ((64, 64), (32, 32), (16, 16))
NQ_ = sum(h * w for h, w in SHAPES_)
NV_ = NQ_
ROWS_ = B_ * NQ_

RB_ = 448
QC_ = 256
VSTRIDE_ = 17

_f32 = jnp.float32
_i32 = jnp.int32


def _lane_consts():
    l_of = np.tile(np.repeat(np.arange(L_), P_), H_)
    wl = np.array([SHAPES_[l][1] for l in l_of])
    hl = np.array([SHAPES_[l][0] for l in l_of])
    bases = np.concatenate([[0], np.cumsum([h * w for h, w in SHAPES_])[:-1]])
    base = np.array([bases[l] for l in l_of])
    return (
        jnp.asarray(wl[None, :], _f32),
        jnp.asarray(hl[None, :], _f32),
        jnp.asarray(1.0 / wl[None, :], _f32),
        jnp.asarray(1.0 / hl[None, :], _f32),
        jnp.asarray((wl - 1)[None, :], _i32),
        jnp.asarray((hl - 1)[None, :], _i32),
        jnp.asarray(base[None, :], _i32),
    )


def _softmax_mat():
    g = np.arange(96) // S_
    return jnp.asarray((g[:, None] == g[None, :]).astype(np.float32))


def _plan_body(q_ref, w_ref, b_ref, m_ref, cw_ref, ch_ref, ciw_ref, cih_ref,
               wmax_ref, hmax_ref, base_ref,
               i00, i01, i10, i11, w00, w01, w10, w11):
    t = jnp.dot(q_ref[...], w_ref[...], preferred_element_type=_f32) + b_ref[...]
    logits = t[:, 0:96]
    offx = t[:, 128:224]
    offy = t[:, 256:352]
    refx = jax.nn.sigmoid(t[:, 384:385])
    refy = jax.nn.sigmoid(t[:, 385:386])

    e = jnp.exp(logits)
    den = jnp.dot(e, m_ref[...], preferred_element_type=_f32)
    aw = e / den

    x = (refx + offx * ciw_ref[...]) * cw_ref[...] - 0.5
    y = (refy + offy * cih_ref[...]) * ch_ref[...] - 0.5
    x0f = jnp.floor(x)
    y0f = jnp.floor(y)
    wx = x - x0f
    wy = y - y0f
    wmax = wmax_ref[...]
    hmax = hmax_ref[...]
    x0 = jnp.clip(x0f.astype(_i32), 0, wmax)
    x1 = jnp.clip(x0f.astype(_i32) + 1, 0, wmax)
    y0 = jnp.clip(y0f.astype(_i32), 0, hmax)
    y1 = jnp.clip(y0f.astype(_i32) + 1, 0, hmax)
    wl_i = wmax + 1
    base = base_ref[...]
    i00[...] = (y0 * wl_i + x0 + base) * VSTRIDE_
    i01[...] = (y0 * wl_i + x1 + base) * VSTRIDE_
    i10[...] = (y1 * wl_i + x0 + base) * VSTRIDE_
    i11[...] = (y1 * wl_i + x1 + base) * VSTRIDE_
    w00[...] = (1.0 - wy) * (1.0 - wx) * aw
    w01[...] = (1.0 - wy) * wx * aw
    w10[...] = wy * (1.0 - wx) * aw
    w11[...] = wy * wx * aw


def _plan_call(q2d, wcat, bcat):
    nb = ROWS_ // RB_
    consts = _lane_consts()
    m = _softmax_mat()
    full = lambda shp: pl.BlockSpec(shp, lambda i: (0, 0))
    out_sh = ([jax.ShapeDtypeStruct((ROWS_, 96), _i32)] * 4
              + [jax.ShapeDtypeStruct((ROWS_, 96), _f32)] * 4)
    return pl.pallas_call(
        _plan_body,
        grid=(nb,),
        in_specs=[pl.BlockSpec((RB_, D_), lambda i: (i, 0)),
                  full((D_, 512)), full((1, 512)), full((96, 96)),
                  full((1, 96)), full((1, 96)), full((1, 96)), full((1, 96)),
                  full((1, 96)), full((1, 96)), full((1, 96))],
        out_specs=[pl.BlockSpec((RB_, 96), lambda i: (i, 0))] * 8,
        out_shape=out_sh,
    )(q2d, wcat, bcat, m, *consts)


def _vproj_body(f_ref, w_ref, b_ref, o_ref):
    o_ref[...] = (jnp.dot(f_ref[...], w_ref[...], preferred_element_type=_f32)
                  + b_ref[...])


def _vproj_call(feat2d, wv, bv):
    nb = ROWS_ // RB_
    return pl.pallas_call(
        _vproj_body,
        grid=(nb,),
        in_specs=[pl.BlockSpec((RB_, D_), lambda i: (i, 0)),
                  pl.BlockSpec((D_, D_), lambda i: (0, 0)),
                  pl.BlockSpec((1, D_), lambda i: (0, 0))],
        out_specs=pl.BlockSpec((RB_, D_), lambda i: (i, 0)),
        out_shape=jax.ShapeDtypeStruct((ROWS_, D_), _f32),
    )(feat2d, wv, bv)


def _sc_gather(vt, idx_a, wgt_a, interpret=False):
    mesh = plsc.VectorSubcoreMesh(core_axis_name="c", subcore_axis_name="s")
    nch = NQ_ // QC_

    @functools.partial(
        pl.kernel,
        out_type=jax.ShapeDtypeStruct((B_, H_, 2, 16, NQ_), _f32),
        mesh=mesh,
        interpret=interpret,
        compiler_params=pltpu.CompilerParams(needs_layout_passes=False),
        scratch_types=[
            pltpu.VMEM((NV_ * VSTRIDE_,), _f32),
            pltpu.VMEM((K_, QC_), _i32),
            pltpu.VMEM((K_, QC_), _f32),
            pltpu.VMEM((16, QC_), _f32),
        ],
    )
    def sc_fn(vt_h, idx_h, wgt_h, out_h, table_v, idx_v, wgt_v, out_v):
        cid = lax.axis_index("c")
        sid = lax.axis_index("s")
        wid = cid * 16 + sid
        b = wid // 16
        r = wid % 16
        h = r // 2
        half = r % 2
        pltpu.sync_copy(vt_h.at[b, h, half], table_v)

        def chunk_body(ci, carry):
            qo = ci * QC_
            pltpu.sync_copy(idx_h.at[b, h, :, pl.ds(qo, QC_)], idx_v)
            pltpu.sync_copy(wgt_h.at[b, h, :, pl.ds(qo, QC_)], wgt_v)

            def g_body(g, c2):
                go = g * 16
                for grp in range(2):
                    accs = [jnp.zeros((16,), _f32) for _ in range(8)]
                    for k in range(K_):
                        iv = idx_v[k, pl.ds(go, 16)]
                        wv = wgt_v[k, pl.ds(go, 16)]
                        for ch in range(8):
                            val = plsc.load_gather(table_v, [iv + (grp * 8 + ch)])
                            accs[ch] = accs[ch] + wv * val
                    for ch in range(8):
                        out_v[grp * 8 + ch, pl.ds(go, 16)] = accs[ch]
                return c2

            lax.fori_loop(0, QC_ // 16, g_body, 0)
            pltpu.sync_copy(out_v, out_h.at[b, h, half, :, pl.ds(qo, QC_)])
            return carry

        lax.fori_loop(0, nch, chunk_body, 0)

    return sc_fn(vt, idx_a, wgt_a)


def _tail_body(a_ref, q_ref, wout_ref, bout_ref, g1_ref, be1_ref,
               w1_ref, b1_ref, w2_ref, b2_ref, g2_ref, be2_ref, o_ref):
    a = (jnp.dot(a_ref[...], wout_ref[...], preferred_element_type=_f32)
         + bout_ref[...] + q_ref[...])
    m = jnp.mean(a, axis=-1, keepdims=True)
    v = jnp.mean((a - m) * (a - m), axis=-1, keepdims=True)
    x1 = (a - m) / jnp.sqrt(v + 1e-5) * g1_ref[...] + be1_ref[...]
    hpre = jnp.dot(x1, w1_ref[...], preferred_element_type=_f32) + b1_ref[...]
    hact = 0.5 * hpre * (1.0 + lax.erf(hpre * np.float32(1.0 / np.sqrt(2.0))))
    yv = jnp.dot(hact, w2_ref[...], preferred_element_type=_f32) + b2_ref[...] + x1
    m2 = jnp.mean(yv, axis=-1, keepdims=True)
    v2 = jnp.mean((yv - m2) * (yv - m2), axis=-1, keepdims=True)
    o_ref[...] = (yv - m2) / jnp.sqrt(v2 + 1e-5) * g2_ref[...] + be2_ref[...]


def _tail_call(attn2d, q2d, wout, bout, g1, be1, w1, b1, w2, b2, g2, be2):
    nb = ROWS_ // RB_
    full = lambda shp: pl.BlockSpec(shp, lambda i: (0, 0))
    return pl.pallas_call(
        _tail_body,
        grid=(nb,),
        in_specs=[pl.BlockSpec((RB_, D_), lambda i: (i, 0)),
                  pl.BlockSpec((RB_, D_), lambda i: (i, 0)),
                  full((D_, D_)), full((1, D_)), full((1, D_)), full((1, D_)),
                  full((D_, DFF_)), full((1, DFF_)),
                  full((DFF_, D_)), full((1, D_)), full((1, D_)), full((1, D_))],
        out_specs=pl.BlockSpec((RB_, D_), lambda i: (i, 0)),
        out_shape=jax.ShapeDtypeStruct((ROWS_, D_), _f32),
    )(attn2d, q2d, wout, bout, g1, be1, w1, b1, w2, b2, g2, be2)


def _build_wcat(Wref, bref, Woff, boff, Wattn, battn):
    wcat = jnp.zeros((D_, 512), _f32)
    bcat = jnp.zeros((1, 512), _f32)
    wcat = wcat.at[:, 0:96].set(Wattn)
    bcat = bcat.at[0, 0:96].set(battn)
    woff_r = Woff.reshape(D_, H_, L_, P_, 2)
    boff_r = boff.reshape(H_, L_, P_, 2)
    wcat = wcat.at[:, 128:224].set(woff_r[..., 0].reshape(D_, 96))
    bcat = bcat.at[0, 128:224].set(boff_r[..., 0].reshape(96))
    wcat = wcat.at[:, 256:352].set(woff_r[..., 1].reshape(D_, 96))
    bcat = bcat.at[0, 256:352].set(boff_r[..., 1].reshape(96))
    wcat = wcat.at[:, 384:386].set(Wref)
    bcat = bcat.at[0, 384:386].set(bref)
    return wcat, bcat


def kernel(query, p3, p4, p5, Wref, bref, Woff, boff, Wattn, battn, Wv, bv,
           Wout, bout, W1, b1, W2, b2, g1, be1, g2, be2):
    q2d = query.reshape(ROWS_, D_)
    wcat, bcat = _build_wcat(Wref, bref, Woff, boff, Wattn, battn)

    plan_out = _plan_call(q2d, wcat, bcat)
    i_corners = plan_out[:4]
    w_corners = plan_out[4:]

    def to_sc(arrs):
        st = jnp.stack(arrs, axis=-1)
        st = st.reshape(B_, NQ_, H_, S_, 4)
        st = st.transpose(0, 2, 3, 4, 1)
        return st.reshape(B_, H_, K_, NQ_)

    idx_a = to_sc(i_corners)
    wgt_a = to_sc(w_corners)

    feat2d = jnp.concatenate([p3, p4, p5], axis=1).reshape(ROWS_, D_)
    v2d = _vproj_call(feat2d, Wv, bv.reshape(1, D_))
    vt = v2d.reshape(B_, NV_, H_, 2, 16).transpose(0, 2, 3, 1, 4)
    vt = jnp.pad(vt, ((0, 0), (0, 0), (0, 0), (0, 0), (0, VSTRIDE_ - 16)))
    vt = vt.reshape(B_, H_, 2, NV_ * VSTRIDE_)

    attn = _sc_gather(vt, idx_a, wgt_a)
    attn2d = attn.transpose(0, 4, 1, 2, 3).reshape(ROWS_, D_)

    out2d = _tail_call(attn2d, q2d, Wout, bout.reshape(1, D_),
                       g1.reshape(1, D_), be1.reshape(1, D_),
                       W1, b1.reshape(1, DFF_), W2, b2.reshape(1, D_),
                       g2.reshape(1, D_), be2.reshape(1, D_))
    return out2d.reshape(B_, NQ_, D_)


# speedup vs baseline: 79.4809x; 1.1496x over previous
import functools

import numpy as np
import jax
import jax.numpy as jnp
from jax import lax
from jax.experimental import pallas as pl
from jax.experimental.pallas import tpu as pltpu
from jax.experimental.pallas import tpu_sc as plsc

B_ = 2
D_ = 256
H_ = 8
HD_ = 32
L_ = 3
P_ = 4
S_ = L_ * P_
K_ = S_ * 4
DFF_ = 1024
SHAPES_ = ((64, 64), (32, 32), (16, 16))
NQ_ = sum(h * w for h, w in SHAPES_)
NV_ = NQ_
ROWS_ = B_ * NQ_

RB_ = 384
QC_ = 256

_f32 = jnp.float32
_i32 = jnp.int32


HS_ = 128


def _lane_consts():
    s_of = np.arange(HS_) % 16
    l_of = np.where(s_of < S_, np.repeat(np.arange(L_), P_)[
        np.minimum(s_of, S_ - 1)], 0)
    wl = np.array([SHAPES_[l][1] for l in l_of])
    hl = np.array([SHAPES_[l][0] for l in l_of])
    bases = np.concatenate([[0], np.cumsum([h * w for h, w in SHAPES_])[:-1]])
    base = np.array([bases[l] for l in l_of])
    return (
        jnp.asarray(wl[:, None], _f32),
        jnp.asarray(hl[:, None], _f32),
        jnp.asarray(1.0 / wl[:, None], _f32),
        jnp.asarray(1.0 / hl[:, None], _f32),
        jnp.asarray((wl - 1)[:, None], _i32),
        jnp.asarray((hl - 1)[:, None], _i32),
        jnp.asarray(base[:, None], _i32),
    )


def _softmax_mat():
    i = np.arange(HS_)
    g = i // 16
    real = (i % 16) < S_
    m = (g[:, None] == g[None, :]) & real[:, None] & real[None, :]
    return jnp.asarray(m.astype(np.float32))


def _dotT(a, b):
    return lax.dot_general(a, b, (((0,), (0,)), ((), ())),
                           preferred_element_type=_f32)


def _plan_body(q_ref, w_ref, b_ref, m_ref, cw_ref, ch_ref, ciw_ref, cih_ref,
               wmax_ref, hmax_ref, base_ref,
               i00, i01, i10, i11, w00, w01, w10, w11):
    t = lax.dot_general(w_ref[...], q_ref[...], (((0,), (1,)), ((), ())),
                        preferred_element_type=_f32) + b_ref[...]
    logits = t[0:128, :]
    offx = t[128:256, :]
    offy = t[256:384, :]
    refx = jax.nn.sigmoid(t[384:385, :])
    refy = jax.nn.sigmoid(t[385:386, :])

    e = jnp.exp(logits)
    den = jnp.dot(m_ref[...], e, preferred_element_type=_f32)
    aw = e / den

    x = (refx + offx * ciw_ref[...]) * cw_ref[...] - 0.5
    y = (refy + offy * cih_ref[...]) * ch_ref[...] - 0.5
    x0f = jnp.floor(x)
    y0f = jnp.floor(y)
    wx = x - x0f
    wy = y - y0f
    wmax = wmax_ref[...]
    hmax = hmax_ref[...]
    x0 = jnp.clip(x0f.astype(_i32), 0, wmax)
    x1 = jnp.clip(x0f.astype(_i32) + 1, 0, wmax)
    y0 = jnp.clip(y0f.astype(_i32), 0, hmax)
    y1 = jnp.clip(y0f.astype(_i32) + 1, 0, hmax)
    wl_i = wmax + 1
    base = base_ref[...]
    one = np.float32(1.0)
    i00[0] = y0 * wl_i + x0 + base
    i01[0] = y0 * wl_i + x1 + base
    i10[0] = y1 * wl_i + x0 + base
    i11[0] = y1 * wl_i + x1 + base
    w00[0] = (one - wy) * (one - wx) * aw
    w01[0] = (one - wy) * wx * aw
    w10[0] = wy * (one - wx) * aw
    w11[0] = wy * wx * aw


def _plan_call(q2d, wcat, bcat):
    nb = ROWS_ // RB_
    consts = _lane_consts()
    m = _softmax_mat()
    full = lambda shp: pl.BlockSpec(shp, lambda i: (0,) * len(shp))
    out_sh = ([jax.ShapeDtypeStruct((B_, HS_, NQ_), _i32)] * 4
              + [jax.ShapeDtypeStruct((B_, HS_, NQ_), _f32)] * 4)
    nbb = nb // B_
    return pl.pallas_call(
        _plan_body,
        grid=(nb,),
        in_specs=[pl.BlockSpec((RB_, D_), lambda i: (i, 0)),
                  full((D_, 512)), full((512, 1)), full((HS_, HS_)),
                  full((HS_, 1)), full((HS_, 1)), full((HS_, 1)),
                  full((HS_, 1)), full((HS_, 1)), full((HS_, 1)),
                  full((HS_, 1))],
        out_specs=[pl.BlockSpec((1, HS_, RB_),
                                lambda i: (i // nbb, 0, i % nbb))] * 8,
        out_shape=out_sh,
    )(q2d, wcat, bcat, m, *consts)


def _vproj_body(f_ref, w_ref, b_ref, o_ref):
    o_ref[0] = lax.dot_general(
        w_ref[...], f_ref[...], (((0,), (1,)), ((), ())),
        preferred_element_type=_f32) + b_ref[...]


def _vproj_call(feat2d, wv, bv_col):
    nb = ROWS_ // RB_
    nbb = nb // B_
    return pl.pallas_call(
        _vproj_body,
        grid=(nb,),
        in_specs=[pl.BlockSpec((RB_, D_), lambda i: (i, 0)),
                  pl.BlockSpec((D_, D_), lambda i: (0, 0)),
                  pl.BlockSpec((D_, 1), lambda i: (0, 0))],
        out_specs=pl.BlockSpec((1, D_, RB_), lambda i: (i // nbb, 0, i % nbb)),
        out_shape=jax.ShapeDtypeStruct((B_, D_, NV_), _f32),
    )(feat2d, wv, bv_col)


def _sc_gather(vt, idxs, wgts):
    mesh = plsc.VectorSubcoreMesh(core_axis_name="c", subcore_axis_name="s")
    nch = NQ_ // QC_

    @functools.partial(
        pl.kernel,
        out_type=jax.ShapeDtypeStruct((B_, D_, NQ_), _f32),
        mesh=mesh,
        compiler_params=pltpu.CompilerParams(needs_layout_passes=False),
        scratch_types=(
            [pltpu.VMEM((16 * NV_,), _f32)]
            + [pltpu.VMEM((16, QC_), _i32) for _ in range(4)]
            + [pltpu.VMEM((16, QC_), _f32) for _ in range(4)]
            + [pltpu.VMEM((16, QC_), _f32)]
        ),
    )
    def sc_fn(vt_h, i0_h, i1_h, i2_h, i3_h, w0_h, w1_h, w2_h, w3_h, out_h,
              table_v, iv0, iv1, iv2, iv3, wv0, wv1, wv2, wv3, out_v):
        cid = lax.axis_index("c")
        sid = lax.axis_index("s")
        wid = cid * 16 + sid
        b = wid // 16
        r = wid % 16
        h = r // 2
        half = r % 2
        chan0 = h * HD_ + half * 16
        for cc in range(16):
            pltpu.sync_copy(vt_h.at[b, chan0 + cc],
                            table_v.at[pl.ds(cc * NV_, NV_)])

        idx_bufs = (iv0, iv1, iv2, iv3)
        wgt_bufs = (wv0, wv1, wv2, wv3)
        idx_srcs = (i0_h, i1_h, i2_h, i3_h)
        wgt_srcs = (w0_h, w1_h, w2_h, w3_h)

        def chunk_body(ci, carry):
            qo = ci * QC_
            for c4 in range(4):
                pltpu.sync_copy(
                    idx_srcs[c4].at[b, pl.ds(h * 16, 16), pl.ds(qo, QC_)],
                    idx_bufs[c4])
                pltpu.sync_copy(
                    wgt_srcs[c4].at[b, pl.ds(h * 16, 16), pl.ds(qo, QC_)],
                    wgt_bufs[c4])

            def g_body(g, c2):
                go = g * 16
                for grp in range(2):
                    accs = [jnp.zeros((16,), _f32) for _ in range(8)]
                    for c4 in range(4):
                        for s in range(S_):
                            iv = idx_bufs[c4][s, pl.ds(go, 16)]
                            wv = wgt_bufs[c4][s, pl.ds(go, 16)]
                            for ch in range(8):
                                cc = grp * 8 + ch
                                val = plsc.load_gather(
                                    table_v, [iv + (cc * NV_)])
                                accs[ch] = accs[ch] + wv * val
                    for ch in range(8):
                        out_v[grp * 8 + ch, pl.ds(go, 16)] = accs[ch]
                return c2

            lax.fori_loop(0, QC_ // 16, g_body, 0)
            pltpu.sync_copy(out_v,
                            out_h.at[b, pl.ds(chan0, 16), pl.ds(qo, QC_)])
            return carry

        lax.fori_loop(0, nch, chunk_body, 0)

    return sc_fn(vt, *idxs, *wgts)


def _tail_body(a_ref, q_ref, wout_ref, bout_ref, g1_ref, be1_ref,
               w1_ref, b1_ref, w2_ref, b2_ref, g2_ref, be2_ref, o_ref):
    a = (_dotT(a_ref[0], wout_ref[...]) + bout_ref[...] + q_ref[...])
    m = jnp.mean(a, axis=-1, keepdims=True)
    v = jnp.mean((a - m) * (a - m), axis=-1, keepdims=True)
    x1 = (a - m) / jnp.sqrt(v + 1e-5) * g1_ref[...] + be1_ref[...]
    hpre = jnp.dot(x1, w1_ref[...], preferred_element_type=_f32) + b1_ref[...]
    hact = 0.5 * hpre * (1.0 + lax.erf(hpre * np.float32(1.0 / np.sqrt(2.0))))
    yv = jnp.dot(hact, w2_ref[...], preferred_element_type=_f32) + b2_ref[...] + x1
    m2 = jnp.mean(yv, axis=-1, keepdims=True)
    v2 = jnp.mean((yv - m2) * (yv - m2), axis=-1, keepdims=True)
    o_ref[...] = (yv - m2) / jnp.sqrt(v2 + 1e-5) * g2_ref[...] + be2_ref[...]


def _tail_call(attn_t, q2d, wout, bout, g1, be1, w1, b1, w2, b2, g2, be2):
    nb = ROWS_ // RB_
    nbb = nb // B_
    full = lambda shp: pl.BlockSpec(shp, lambda i: (0,) * len(shp))
    return pl.pallas_call(
        _tail_body,
        grid=(nb,),
        in_specs=[pl.BlockSpec((1, D_, RB_), lambda i: (i // nbb, 0, i % nbb)),
                  pl.BlockSpec((RB_, D_), lambda i: (i, 0)),
                  full((D_, D_)), full((1, D_)), full((1, D_)), full((1, D_)),
                  full((D_, DFF_)), full((1, DFF_)),
                  full((DFF_, D_)), full((1, D_)), full((1, D_)), full((1, D_))],
        out_specs=pl.BlockSpec((RB_, D_), lambda i: (i, 0)),
        out_shape=jax.ShapeDtypeStruct((ROWS_, D_), _f32),
    )(attn_t, q2d, wout, bout, g1, be1, w1, b1, w2, b2, g2, be2)


def _build_wcat(Wref, bref, Woff, boff, Wattn, battn):
    def pad_hs(w):
        wp = jnp.zeros((D_, H_, 16), _f32).at[:, :, :S_].set(w)
        return wp.reshape(D_, HS_)

    def pad_hs_b(bvec):
        bp = jnp.zeros((H_, 16), _f32).at[:, :S_].set(bvec)
        return bp.reshape(HS_)

    wcat = jnp.zeros((D_, 512), _f32)
    bcat = jnp.zeros((512, 1), _f32)
    wcat = wcat.at[:, 0:128].set(pad_hs(Wattn.reshape(D_, H_, S_)))
    bcat = bcat.at[0:128, 0].set(pad_hs_b(battn.reshape(H_, S_)))
    woff_r = Woff.reshape(D_, H_, L_ * P_, 2)
    boff_r = boff.reshape(H_, L_ * P_, 2)
    wcat = wcat.at[:, 128:256].set(pad_hs(woff_r[..., 0]))
    bcat = bcat.at[128:256, 0].set(pad_hs_b(boff_r[..., 0]))
    wcat = wcat.at[:, 256:384].set(pad_hs(woff_r[..., 1]))
    bcat = bcat.at[256:384, 0].set(pad_hs_b(boff_r[..., 1]))
    wcat = wcat.at[:, 384:386].set(Wref)
    bcat = bcat.at[384:386, 0].set(bref)
    return wcat, bcat


def kernel(query, p3, p4, p5, Wref, bref, Woff, boff, Wattn, battn, Wv, bv,
           Wout, bout, W1, b1, W2, b2, g1, be1, g2, be2):
    q2d = query.reshape(ROWS_, D_)
    wcat, bcat = _build_wcat(Wref, bref, Woff, boff, Wattn, battn)

    plan_out = _plan_call(q2d, wcat, bcat)
    idxs = plan_out[:4]
    wgts = plan_out[4:]

    feat2d = jnp.concatenate([p3, p4, p5], axis=1).reshape(ROWS_, D_)
    vt = _vproj_call(feat2d, Wv, bv.reshape(D_, 1))

    attn_t = _sc_gather(vt, idxs, wgts)

    out2d = _tail_call(attn_t, q2d, Wout, bout.reshape(1, D_),
                       g1.reshape(1, D_), be1.reshape(1, D_),
                       W1, b1.reshape(1, DFF_), W2, b2.reshape(1, D_),
                       g2.reshape(1, D_), be2.reshape(1, D_))
    return out2d.reshape(B_, NQ_, D_)
